# baseline (speedup 1.0000x reference)
import functools

import jax
import jax.numpy as jnp
from jax import lax
from jax.experimental import pallas as pl
from jax.experimental.pallas import tpu as pltpu
from jax.experimental.pallas import tpu_sc as plsc

K_EMB = 1024
D_DIM = 256
N_TOK = 16384
SPLITS = 2
N_PART = N_TOK // SPLITS
ROWS = 512
GRID1 = N_PART // ROWS

SC_CORES = 2
SC_SUBCORES = 16
SC_WORKERS = SC_CORES * SC_SUBCORES
ROWS_PER_WORKER = N_PART // SC_WORKERS
SC_CHUNK = 64
SC_NCHUNKS = ROWS_PER_WORKER // SC_CHUNK
SC_NBUF = 4


def _argmin_body(x_ref, w_ref, idx_ref, cnt_ref, esum_ref, wn_ref):
    step = pl.program_id(0)

    @pl.when(step == 0)
    def _():
        w0 = w_ref[...]
        wn_ref[...] = lax.dot_general(
            jnp.ones((1, D_DIM), jnp.float32), w0 * w0,
            (((1,), (1,)), ((), ())),
            precision=lax.Precision.HIGHEST)

    x = x_ref[...]
    w = w_ref[...]
    x2 = jnp.sum(x * x, axis=1, keepdims=True)
    wn = wn_ref[...]
    mm = lax.dot_general(x, w, (((1,), (1,)), ((), ())),
                         preferred_element_type=jnp.float32)
    d = (x2 + wn) - 2.0 * mm
    m = jnp.min(d, axis=1, keepdims=True)
    col = lax.broadcasted_iota(jnp.int32, (ROWS, K_EMB), 1)
    idx = jnp.min(jnp.where(d == m, col, K_EMB), axis=1, keepdims=True)
    idx_ref[...] = idx
    part = jnp.sum((col == idx).astype(jnp.float32), axis=0, keepdims=True)
    es = jnp.sum(m, axis=0, keepdims=True)

    @pl.when(step == 0)
    def _():
        cnt_ref[...] = part
        esum_ref[...] = es

    @pl.when(step != 0)
    def _():
        cnt_ref[...] += part
        esum_ref[...] += es


def _argmin_call(flat_part, w):
    return pl.pallas_call(
        _argmin_body,
        grid=(GRID1,),
        in_specs=[
            pl.BlockSpec((ROWS, D_DIM), lambda i: (i, 0)),
            pl.BlockSpec((K_EMB, D_DIM), lambda i: (0, 0)),
        ],
        out_specs=[
            pl.BlockSpec((ROWS, 1), lambda i: (i, 0)),
            pl.BlockSpec((1, K_EMB), lambda i: (0, 0)),
            pl.BlockSpec((1, 1), lambda i: (0, 0)),
        ],
        out_shape=[
            jax.ShapeDtypeStruct((N_PART, 1), jnp.int32),
            jax.ShapeDtypeStruct((1, K_EMB), jnp.float32),
            jax.ShapeDtypeStruct((1, 1), jnp.float32),
        ],
        scratch_shapes=[pltpu.VMEM((1, K_EMB), jnp.float32)],
        compiler_params=pltpu.CompilerParams(
            dimension_semantics=("arbitrary",)),
    )(flat_part, w)


def _loss_body(w_ref, b_ref, cnt_ref, esum_ref, loss_ref, perp_ref):
    w = w_ref[...]
    b = b_ref[...]
    counts = jnp.sum(cnt_ref[...], axis=0, keepdims=True)
    esum = jnp.sum(esum_ref[...], axis=0, keepdims=True)
    wn_col = jnp.sum(w * w, axis=1, keepdims=True)
    bw = lax.dot_general(b, w, (((1,), (0,)), ((), ())),
                         preferred_element_type=jnp.float32)
    t1 = lax.dot_general(b, wn_col, (((1,), (0,)), ((), ())),
                         preferred_element_type=jnp.float32)
    t2 = wn_col * jnp.sum(b, axis=1, keepdims=True)
    t3 = jnp.sum(w * bw, axis=1, keepdims=True)
    s = t1 + t2 - 2.0 * t3
    ksum = lax.dot_general(counts, s, (((1,), (0,)), ((), ())),
                           preferred_element_type=jnp.float32)
    inv_n = 1.0 / N_TOK
    loss_ref[...] = (esum * (1.25 / (N_TOK * D_DIM))
                     + ksum * inv_n)
    p = counts * inv_n
    ent = jnp.sum(p * jnp.log(p + 1e-10), axis=1, keepdims=True)
    perp_ref[...] = jnp.exp(-ent)


@functools.cache
def _make_sc_gather():
    mesh = plsc.VectorSubcoreMesh(core_axis_name="c", subcore_axis_name="s")

    @functools.partial(
        pl.kernel,
        out_type=jax.ShapeDtypeStruct((N_PART, D_DIM), jnp.float32),
        mesh=mesh,
        scratch_types=(
            [pltpu.VMEM((ROWS_PER_WORKER,), jnp.int32)]
            + [pltpu.VMEM((SC_CHUNK, D_DIM), jnp.float32)] * SC_NBUF
            + [pltpu.SemaphoreType.DMA] * (2 * SC_NBUF)
        ),
    )
    def _sc_gather(w_hbm, idx_hbm, out_hbm, idx_v, *rest):
        bufs = rest[:SC_NBUF]
        gsems = rest[SC_NBUF:2 * SC_NBUF]
        wsems = rest[2 * SC_NBUF:3 * SC_NBUF]
        wid = lax.axis_index("s") * SC_CORES + lax.axis_index("c")
        base = wid * ROWS_PER_WORKER
        pltpu.sync_copy(idx_hbm.at[pl.ds(base, ROWS_PER_WORKER)], idx_v)

        def gather(c):
            b = c % SC_NBUF
            return pltpu.async_copy(
                w_hbm.at[idx_v.at[pl.ds(c * SC_CHUNK, SC_CHUNK)]],
                bufs[b], gsems[b])

        def write(c):
            b = c % SC_NBUF
            return pltpu.async_copy(
                bufs[b], out_hbm.at[pl.ds(base + c * SC_CHUNK, SC_CHUNK)],
                wsems[b])

        g = {c: gather(c) for c in range(min(SC_NBUF, SC_NCHUNKS))}
        wr = {}
        for c in range(SC_NCHUNKS):
            g[c].wait()
            wr[c] = write(c)
            nxt = c + SC_NBUF
            if nxt < SC_NCHUNKS:
                wr[c].wait()
                g[nxt] = gather(nxt)
        for c in range(max(0, SC_NCHUNKS - SC_NBUF), SC_NCHUNKS):
            wr[c].wait()

    return _sc_gather


def kernel(inputs, w, bmat):
    flat = inputs.reshape(-1, D_DIM)
    sc_gather = _make_sc_gather()
    idxs, cnts, ess, qs = [], [], [], []
    for i in range(SPLITS):
        part = lax.slice_in_dim(flat, i * N_PART, (i + 1) * N_PART, axis=0)
        idx2d, cnt, es = _argmin_call(part, w)
        idxs.append(idx2d)
        cnts.append(cnt)
        ess.append(es)
        qs.append(sc_gather(w, idx2d.reshape(-1)))

    loss2d, perp2d = pl.pallas_call(
        _loss_body,
        out_shape=[
            jax.ShapeDtypeStruct((1, 1), jnp.float32),
            jax.ShapeDtypeStruct((1, 1), jnp.float32),
        ],
    )(w, bmat, jnp.concatenate(cnts, axis=0), jnp.concatenate(ess, axis=0))

    quantized = jnp.concatenate(qs, axis=0).reshape(inputs.shape)
    idx_all = jnp.concatenate(idxs, axis=0)
    return loss2d[0, 0], quantized, perp2d[0, 0], idx_all


# speedup vs baseline: 1.2027x; 1.2027x over previous
import functools

import jax
import jax.numpy as jnp
from jax import lax
from jax.experimental import pallas as pl
from jax.experimental.pallas import tpu as pltpu
from jax.experimental.pallas import tpu_sc as plsc

K_EMB = 1024
D_DIM = 256
N_TOK = 16384
SPLITS = 1
N_PART = N_TOK // SPLITS
ROWS = 1024
GRID1 = N_PART // ROWS

SC_CORES = 2
SC_SUBCORES = 16
SC_WORKERS = SC_CORES * SC_SUBCORES
ROWS_PER_WORKER = N_PART // SC_WORKERS
SC_CHUNK = 64
SC_NCHUNKS = ROWS_PER_WORKER // SC_CHUNK
SC_NBUF = 4


def _argmin_body(x_ref, w_ref, idx_ref, cnt_ref, esum_ref, wn_ref):
    step = pl.program_id(0)

    @pl.when(step == 0)
    def _():
        w0 = w_ref[...]
        wn_ref[...] = lax.dot_general(
            jnp.ones((1, D_DIM), jnp.float32), w0 * w0,
            (((1,), (1,)), ((), ())),
            precision=lax.Precision.HIGHEST)

    x = x_ref[...]
    w = w_ref[...]
    x2 = jnp.sum(x * x, axis=1, keepdims=True)
    wn = wn_ref[...]
    mm = lax.dot_general(x, w, (((1,), (1,)), ((), ())),
                         preferred_element_type=jnp.float32)
    d = (x2 + wn) - 2.0 * mm
    m = jnp.min(d, axis=1, keepdims=True)
    col = lax.broadcasted_iota(jnp.int32, (ROWS, K_EMB), 1)
    idx = jnp.min(jnp.where(d == m, col, K_EMB), axis=1, keepdims=True)
    idx_ref[...] = idx
    part = jnp.sum((col == idx).astype(jnp.float32), axis=0, keepdims=True)
    es = jnp.sum(m, axis=0, keepdims=True)

    @pl.when(step == 0)
    def _():
        cnt_ref[...] = part
        esum_ref[...] = es

    @pl.when(step != 0)
    def _():
        cnt_ref[...] += part
        esum_ref[...] += es


def _argmin_call(flat_part, w):
    return pl.pallas_call(
        _argmin_body,
        grid=(GRID1,),
        in_specs=[
            pl.BlockSpec((ROWS, D_DIM), lambda i: (i, 0)),
            pl.BlockSpec((K_EMB, D_DIM), lambda i: (0, 0)),
        ],
        out_specs=[
            pl.BlockSpec((ROWS, 1), lambda i: (i, 0)),
            pl.BlockSpec((1, K_EMB), lambda i: (0, 0)),
            pl.BlockSpec((1, 1), lambda i: (0, 0)),
        ],
        out_shape=[
            jax.ShapeDtypeStruct((N_PART, 1), jnp.int32),
            jax.ShapeDtypeStruct((1, K_EMB), jnp.float32),
            jax.ShapeDtypeStruct((1, 1), jnp.float32),
        ],
        scratch_shapes=[pltpu.VMEM((1, K_EMB), jnp.float32)],
        compiler_params=pltpu.CompilerParams(
            dimension_semantics=("arbitrary",)),
    )(flat_part, w)


def _loss_body(w_ref, b_ref, cnt_ref, esum_ref, loss_ref, perp_ref):
    w = w_ref[...]
    b = b_ref[...]
    counts = jnp.sum(cnt_ref[...], axis=0, keepdims=True)
    esum = jnp.sum(esum_ref[...], axis=0, keepdims=True)
    wn_col = jnp.sum(w * w, axis=1, keepdims=True)
    bw = lax.dot_general(b, w, (((1,), (0,)), ((), ())),
                         preferred_element_type=jnp.float32)
    t1 = lax.dot_general(b, wn_col, (((1,), (0,)), ((), ())),
                         preferred_element_type=jnp.float32)
    t2 = wn_col * jnp.sum(b, axis=1, keepdims=True)
    t3 = jnp.sum(w * bw, axis=1, keepdims=True)
    s = t1 + t2 - 2.0 * t3
    ksum = lax.dot_general(counts, s, (((1,), (0,)), ((), ())),
                           preferred_element_type=jnp.float32)
    inv_n = 1.0 / N_TOK
    loss_ref[...] = (esum * (1.25 / (N_TOK * D_DIM))
                     + ksum * inv_n)
    p = counts * inv_n
    ent = jnp.sum(p * jnp.log(p + 1e-10), axis=1, keepdims=True)
    perp_ref[...] = jnp.exp(-ent)


@functools.cache
def _make_sc_gather():
    mesh = plsc.VectorSubcoreMesh(core_axis_name="c", subcore_axis_name="s")

    @functools.partial(
        pl.kernel,
        out_type=jax.ShapeDtypeStruct((N_PART, D_DIM), jnp.float32),
        mesh=mesh,
        scratch_types=(
            [pltpu.VMEM((ROWS_PER_WORKER,), jnp.int32)]
            + [pltpu.VMEM((SC_CHUNK, D_DIM), jnp.float32)] * SC_NBUF
            + [pltpu.SemaphoreType.DMA] * (2 * SC_NBUF)
        ),
    )
    def _sc_gather(w_hbm, idx_hbm, out_hbm, idx_v, *rest):
        bufs = rest[:SC_NBUF]
        gsems = rest[SC_NBUF:2 * SC_NBUF]
        wsems = rest[2 * SC_NBUF:3 * SC_NBUF]
        wid = lax.axis_index("s") * SC_CORES + lax.axis_index("c")
        base = wid * ROWS_PER_WORKER
        pltpu.sync_copy(idx_hbm.at[pl.ds(base, ROWS_PER_WORKER)], idx_v)

        def gather(c):
            b = c % SC_NBUF
            return pltpu.async_copy(
                w_hbm.at[idx_v.at[pl.ds(c * SC_CHUNK, SC_CHUNK)]],
                bufs[b], gsems[b])

        def write(c):
            b = c % SC_NBUF
            return pltpu.async_copy(
                bufs[b], out_hbm.at[pl.ds(base + c * SC_CHUNK, SC_CHUNK)],
                wsems[b])

        g = {c: gather(c) for c in range(min(SC_NBUF, SC_NCHUNKS))}
        wr = {}
        for c in range(SC_NCHUNKS):
            g[c].wait()
            wr[c] = write(c)
            nxt = c + SC_NBUF
            if nxt < SC_NCHUNKS:
                wr[c].wait()
                g[nxt] = gather(nxt)
        for c in range(max(0, SC_NCHUNKS - SC_NBUF), SC_NCHUNKS):
            wr[c].wait()

    return _sc_gather


def kernel(inputs, w, bmat):
    flat = inputs.reshape(-1, D_DIM)
    sc_gather = _make_sc_gather()
    idxs, cnts, ess, qs = [], [], [], []
    for i in range(SPLITS):
        part = lax.slice_in_dim(flat, i * N_PART, (i + 1) * N_PART, axis=0)
        idx2d, cnt, es = _argmin_call(part, w)
        idxs.append(idx2d)
        cnts.append(cnt)
        ess.append(es)
        qs.append(sc_gather(w, idx2d.reshape(-1)))

    loss2d, perp2d = pl.pallas_call(
        _loss_body,
        out_shape=[
            jax.ShapeDtypeStruct((1, 1), jnp.float32),
            jax.ShapeDtypeStruct((1, 1), jnp.float32),
        ],
    )(w, bmat, jnp.concatenate(cnts, axis=0), jnp.concatenate(ess, axis=0))

    quantized = jnp.concatenate(qs, axis=0).reshape(inputs.shape)
    idx_all = jnp.concatenate(idxs, axis=0)
    return loss2d[0, 0], quantized, perp2d[0, 0], idx_all


# speedup vs baseline: 1.2237x; 1.0174x over previous
import functools

import jax
import jax.numpy as jnp
from jax import lax
from jax.experimental import pallas as pl
from jax.experimental.pallas import tpu as pltpu
from jax.experimental.pallas import tpu_sc as plsc

K_EMB = 1024
D_DIM = 256
N_TOK = 16384
SPLITS = 1
N_PART = N_TOK // SPLITS
ROWS = 2048
GRID1 = N_PART // ROWS

SC_CORES = 2
SC_SUBCORES = 16
SC_WORKERS = SC_CORES * SC_SUBCORES
ROWS_PER_WORKER = N_PART // SC_WORKERS
SC_CHUNK = 64
SC_NCHUNKS = ROWS_PER_WORKER // SC_CHUNK
SC_NBUF = 4


def _argmin_body(x_ref, w_ref, idx_ref, cnt_ref, esum_ref, wn_ref):
    step = pl.program_id(0)

    @pl.when(step == 0)
    def _():
        w0 = w_ref[...]
        wn_ref[...] = lax.dot_general(
            jnp.ones((1, D_DIM), jnp.float32), w0 * w0,
            (((1,), (1,)), ((), ())),
            precision=lax.Precision.HIGHEST)

    x = x_ref[...]
    w = w_ref[...]
    x2 = jnp.sum(x * x, axis=1, keepdims=True)
    wn = wn_ref[...]
    mm = lax.dot_general(x, w, (((1,), (1,)), ((), ())),
                         preferred_element_type=jnp.float32)
    d = (x2 + wn) - 2.0 * mm
    m = jnp.min(d, axis=1, keepdims=True)
    col = lax.broadcasted_iota(jnp.int32, (ROWS, K_EMB), 1)
    idx = jnp.min(jnp.where(d == m, col, K_EMB), axis=1, keepdims=True)
    idx_ref[...] = idx
    part = jnp.sum((col == idx).astype(jnp.float32), axis=0, keepdims=True)
    es = jnp.sum(m, axis=0, keepdims=True)

    @pl.when(step == 0)
    def _():
        cnt_ref[...] = part
        esum_ref[...] = es

    @pl.when(step != 0)
    def _():
        cnt_ref[...] += part
        esum_ref[...] += es


def _argmin_call(flat_part, w):
    return pl.pallas_call(
        _argmin_body,
        grid=(GRID1,),
        in_specs=[
            pl.BlockSpec((ROWS, D_DIM), lambda i: (i, 0)),
            pl.BlockSpec((K_EMB, D_DIM), lambda i: (0, 0)),
        ],
        out_specs=[
            pl.BlockSpec((ROWS, 1), lambda i: (i, 0)),
            pl.BlockSpec((1, K_EMB), lambda i: (0, 0)),
            pl.BlockSpec((1, 1), lambda i: (0, 0)),
        ],
        out_shape=[
            jax.ShapeDtypeStruct((N_PART, 1), jnp.int32),
            jax.ShapeDtypeStruct((1, K_EMB), jnp.float32),
            jax.ShapeDtypeStruct((1, 1), jnp.float32),
        ],
        scratch_shapes=[pltpu.VMEM((1, K_EMB), jnp.float32)],
        compiler_params=pltpu.CompilerParams(
            dimension_semantics=("arbitrary",)),
    )(flat_part, w)


def _loss_body(w_ref, b_ref, cnt_ref, esum_ref, loss_ref, perp_ref):
    w = w_ref[...]
    b = b_ref[...]
    counts = jnp.sum(cnt_ref[...], axis=0, keepdims=True)
    esum = jnp.sum(esum_ref[...], axis=0, keepdims=True)
    wn_col = jnp.sum(w * w, axis=1, keepdims=True)
    bw = lax.dot_general(b, w, (((1,), (0,)), ((), ())),
                         preferred_element_type=jnp.float32)
    t1 = lax.dot_general(b, wn_col, (((1,), (0,)), ((), ())),
                         preferred_element_type=jnp.float32)
    t2 = wn_col * jnp.sum(b, axis=1, keepdims=True)
    t3 = jnp.sum(w * bw, axis=1, keepdims=True)
    s = t1 + t2 - 2.0 * t3
    ksum = lax.dot_general(counts, s, (((1,), (0,)), ((), ())),
                           preferred_element_type=jnp.float32)
    inv_n = 1.0 / N_TOK
    loss_ref[...] = (esum * (1.25 / (N_TOK * D_DIM))
                     + ksum * inv_n)
    p = counts * inv_n
    ent = jnp.sum(p * jnp.log(p + 1e-10), axis=1, keepdims=True)
    perp_ref[...] = jnp.exp(-ent)


@functools.cache
def _make_sc_gather():
    mesh = plsc.VectorSubcoreMesh(core_axis_name="c", subcore_axis_name="s")

    @functools.partial(
        pl.kernel,
        out_type=jax.ShapeDtypeStruct((N_PART, D_DIM), jnp.float32),
        mesh=mesh,
        scratch_types=(
            [pltpu.VMEM((ROWS_PER_WORKER,), jnp.int32)]
            + [pltpu.VMEM((SC_CHUNK, D_DIM), jnp.float32)] * SC_NBUF
            + [pltpu.SemaphoreType.DMA] * (2 * SC_NBUF)
        ),
    )
    def _sc_gather(w_hbm, idx_hbm, out_hbm, idx_v, *rest):
        bufs = rest[:SC_NBUF]
        gsems = rest[SC_NBUF:2 * SC_NBUF]
        wsems = rest[2 * SC_NBUF:3 * SC_NBUF]
        wid = lax.axis_index("s") * SC_CORES + lax.axis_index("c")
        base = wid * ROWS_PER_WORKER
        pltpu.sync_copy(idx_hbm.at[pl.ds(base, ROWS_PER_WORKER)], idx_v)

        def gather(c):
            b = c % SC_NBUF
            return pltpu.async_copy(
                w_hbm.at[idx_v.at[pl.ds(c * SC_CHUNK, SC_CHUNK)]],
                bufs[b], gsems[b])

        def write(c):
            b = c % SC_NBUF
            return pltpu.async_copy(
                bufs[b], out_hbm.at[pl.ds(base + c * SC_CHUNK, SC_CHUNK)],
                wsems[b])

        g = {c: gather(c) for c in range(min(SC_NBUF, SC_NCHUNKS))}
        wr = {}
        for c in range(SC_NCHUNKS):
            g[c].wait()
            wr[c] = write(c)
            nxt = c + SC_NBUF
            if nxt < SC_NCHUNKS:
                wr[c].wait()
                g[nxt] = gather(nxt)
        for c in range(max(0, SC_NCHUNKS - SC_NBUF), SC_NCHUNKS):
            wr[c].wait()

    return _sc_gather


def kernel(inputs, w, bmat):
    flat = inputs.reshape(-1, D_DIM)
    sc_gather = _make_sc_gather()
    idxs, cnts, ess, qs = [], [], [], []
    for i in range(SPLITS):
        part = lax.slice_in_dim(flat, i * N_PART, (i + 1) * N_PART, axis=0)
        idx2d, cnt, es = _argmin_call(part, w)
        idxs.append(idx2d)
        cnts.append(cnt)
        ess.append(es)
        qs.append(sc_gather(w, idx2d.reshape(-1)))

    loss2d, perp2d = pl.pallas_call(
        _loss_body,
        out_shape=[
            jax.ShapeDtypeStruct((1, 1), jnp.float32),
            jax.ShapeDtypeStruct((1, 1), jnp.float32),
        ],
    )(w, bmat, jnp.concatenate(cnts, axis=0), jnp.concatenate(ess, axis=0))

    quantized = jnp.concatenate(qs, axis=0).reshape(inputs.shape)
    idx_all = jnp.concatenate(idxs, axis=0)
    return loss2d[0, 0], quantized, perp2d[0, 0], idx_all
